# Initial kernel scaffold; baseline (speedup 1.0000x reference)
import jax
import jax.numpy as jnp
from jax.experimental import pallas as pl


def kernel(node, edge, edge_index, node_batch_index, coupling_index, coupling_type, coupling_batch_index, W_pre, g_pre, b_pre, We1, ge1, be1, We2, ge2, be2, W_root, b_root, Wih, Whh, bih, bhh, Wi, Wh, bi, bh, W_pred, g_pred, b_pred):
    raise NotImplementedError("write your pallas kernel here")


# speedup vs baseline: 1.1914x; 1.1914x over previous
import functools

import jax
import jax.numpy as jnp
from jax import lax
from jax.experimental import pallas as pl
from jax.experimental.pallas import tpu as pltpu
from jax.experimental.pallas import tpu_sc as plsc

N = 10000; E = 16000; B = 500; C = 20000
H = 64; ND = 13; ED = 5; NT = 8; STEPS = 3
EPS = 1e-5

NC, NS, NWK = 2, 16, 32
EP = 16384
NPAD = 10240
BP = 512
CP = 20480
C2P = 40960
F32 = jnp.float32
_PHI = lax.Precision.HIGHEST


def _ddot(a, b):
    return jnp.dot(a.astype(jnp.bfloat16), b.astype(jnp.bfloat16),
                   preferred_element_type=F32)


def _prep1_body(node_ref, wpre_ref, gpre_ref, bpre_ref,
                edge_ref, we1_ref, ge1_ref, be1_ref, x0_ref, e1_ref):
    y0 = _ddot(node_ref[...], wpre_ref[...])
    mu0 = jnp.sum(y0, 0, keepdims=True) / N
    d0 = y0 - mu0
    var0 = jnp.sum(d0 * d0, 0, keepdims=True) / N
    x0 = jnp.maximum(
        gpre_ref[...] * d0 * lax.rsqrt(var0 + EPS) + bpre_ref[...], 0.0)
    x0_ref[...] = jnp.concatenate([x0, jnp.zeros((N, H), F32)], axis=1)

    y1 = _ddot(edge_ref[...], we1_ref[...])
    mu1 = jnp.sum(y1, 0, keepdims=True) / E
    var1 = jnp.sum(y1 * y1, 0, keepdims=True) / E - mu1 * mu1
    e1 = jnp.maximum(
        ge1_ref[...] * (y1 - mu1) * lax.rsqrt(var1 + EPS) + be1_ref[...], 0.0)
    rid = lax.broadcasted_iota(jnp.int32, (EP, H), 0)
    e1_ref[...] = jnp.where(rid < E, e1, 0.0)


def _prep1(node_p, wpre_p, gpre, bpre, edge_p, we1_p, ge1, be1):
    return pl.pallas_call(
        _prep1_body,
        out_shape=(jax.ShapeDtypeStruct((N, 2 * H), F32),
                   jax.ShapeDtypeStruct((EP, H), F32)),
    )(node_p, wpre_p, gpre, bpre, edge_p, we1_p, ge1, be1)


_TE = 256


def _esum_body(e1_ref, we2_ref, sum_ref, esum_ref):
    i = pl.program_id(0)
    y = _ddot(e1_ref[...], we2_ref[...])

    @pl.when(i == 0)
    def _():
        sum_ref[...] = jnp.zeros_like(sum_ref)
        esum_ref[...] = jnp.zeros_like(esum_ref)

    sum_ref[...] += jnp.sum(y, 0, keepdims=True)
    esum_ref[...] += jnp.sum(e1_ref[...], 0, keepdims=True)


def _essq_body(e1_ref, we2_ref, sum_ref, ssq_ref):
    i = pl.program_id(0)
    y = _ddot(e1_ref[...], we2_ref[...])
    row = i * _TE + lax.broadcasted_iota(jnp.int32, (_TE, H * H), 0)
    dev = jnp.where(row < E, y - sum_ref[...] / E, 0.0)

    @pl.when(i == 0)
    def _():
        ssq_ref[...] = jnp.zeros_like(ssq_ref)

    ssq_ref[...] += jnp.sum(dev * dev, 0, keepdims=True)


def _prep2_body(sum_ref, ssq_ref, ge2_ref, mu_ref, sc_ref):
    mu2 = sum_ref[...] / E
    var2 = ssq_ref[...] / E
    mu_ref[...] = mu2
    sc_ref[...] = ge2_ref[...] / jnp.sqrt(var2 + EPS)


def _prep2(e1, we2, ge2):
    espec = pl.BlockSpec((_TE, H), lambda i: (i, 0))
    wspec = pl.BlockSpec((H, H * H), lambda i: (0, 0))
    sspec = pl.BlockSpec((1, H * H), lambda i: (0, 0))
    sums, _ = pl.pallas_call(
        _esum_body,
        grid=(EP // _TE,),
        in_specs=[espec, wspec],
        out_specs=(sspec, pl.BlockSpec((1, H), lambda i: (0, 0))),
        out_shape=(jax.ShapeDtypeStruct((1, H * H), F32),
                   jax.ShapeDtypeStruct((1, H), F32)),
    )(e1, we2)
    ssq = pl.pallas_call(
        _essq_body,
        grid=(EP // _TE,),
        in_specs=[espec, wspec, sspec],
        out_specs=sspec,
        out_shape=jax.ShapeDtypeStruct((1, H * H), F32),
    )(e1, we2, sums)
    return pl.pallas_call(
        _prep2_body,
        out_shape=(jax.ShapeDtypeStruct((1, H * H), F32),
                   jax.ShapeDtypeStruct((1, H * H), F32)),
    )(sums, ssq, ge2)


_TM = 256


def _msg_body(xs_ref, e1_ref, we2_ref, mu_ref, sc_ref, be2_ref, msg_ref):
    y = _ddot(e1_ref[...], we2_ref[...])
    w = (y - mu_ref[...]) * sc_ref[...] + be2_ref[...]
    xs = xs_ref[:, :H]
    m = xs[:, 0:1] * w[:, 0:H]
    for h in range(1, H):
        m = m + xs[:, h:h + 1] * w[:, h * H:(h + 1) * H]
    msg_ref[...] = jnp.concatenate(
        [m, jnp.ones((_TM, 1), F32), jnp.zeros((_TM, H - 1), F32)], axis=1)


def _msg(xs, e1, we2, mu2, sc2, be2):
    grid = (EP // _TM,)
    rspec = pl.BlockSpec((1, H * H), lambda i: (0, 0))
    return pl.pallas_call(
        _msg_body,
        grid=grid,
        in_specs=[
            pl.BlockSpec((_TM, 2 * H), lambda i: (i, 0)),
            pl.BlockSpec((_TM, H), lambda i: (i, 0)),
            pl.BlockSpec((H, H * H), lambda i: (0, 0)),
            rspec, rspec, rspec,
        ],
        out_specs=pl.BlockSpec((_TM, 2 * H), lambda i: (i, 0)),
        out_shape=jax.ShapeDtypeStruct((EP, 2 * H), F32),
    )(xs, e1, we2, mu2, sc2, be2)


def _gru_body(x_ref, p0_ref, p1_ref,
              wroot_ref, broot_ref, wih_ref, whh_ref, bih_ref, bhh_ref,
              out_ref):
    x = x_ref[:, :H]
    p = p0_ref[...] + p1_ref[...]
    deg = p[:, H:H + 1]
    aggr = p[:, :H] / jnp.maximum(deg, 1.0)
    m = jnp.maximum(
        aggr + _ddot(x, wroot_ref[...])
        + broot_ref[...], 0.0)
    gi = _ddot(m, wih_ref[...]) + bih_ref[...]
    gh = _ddot(x, whh_ref[...]) + bhh_ref[...]
    r = jax.nn.sigmoid(gi[:, :H] + gh[:, :H])
    z = jax.nn.sigmoid(gi[:, H:2 * H] + gh[:, H:2 * H])
    n = jnp.tanh(gi[:, 2 * H:] + r * gh[:, 2 * H:])
    h = (1.0 - z) * n + z * x
    out_ref[...] = jnp.concatenate([h, jnp.zeros((N, H), F32)], axis=1)


def _gru(x, p0, p1, wroot, broot, wih, whh, bih, bhh):
    return pl.pallas_call(
        _gru_body,
        out_shape=jax.ShapeDtypeStruct((N, 2 * H), F32),
    )(x, p0, p1, wroot, broot, wih, whh, bih, bhh)


N2 = 10240
_CH = 1280
_NCHUNK = N2 // _CH


def _s2s_body(x_ref, nbc_ref, nbr_ref, wi_ref, wh_ref, bi_ref, bh_ref,
              qs_ref, eatt_ref):
    q_star = jnp.zeros((BP, 2 * H), F32)
    hS = jnp.zeros((BP, H), F32)
    cS = jnp.zeros((BP, H), F32)
    iota_row = lax.broadcasted_iota(jnp.int32, (_CH, BP), 1)
    iota_col = lax.broadcasted_iota(jnp.int32, (BP, _CH), 0)
    for _ in range(STEPS):
        gates = (_ddot(q_star, wi_ref[...]) + bi_ref[...]
                 + _ddot(hS, wh_ref[...]) + bh_ref[...])
        i_g = jax.nn.sigmoid(gates[:, :H])
        f_g = jax.nn.sigmoid(gates[:, H:2 * H])
        g_g = jnp.tanh(gates[:, 2 * H:3 * H])
        o_g = jax.nn.sigmoid(gates[:, 3 * H:])
        cS = f_g * cS + i_g * g_g
        hS = o_g * jnp.tanh(cS)
        q = hS

        def pass1(i, emax_row):
            xc = x_ref[pl.ds(i * _CH, _CH), :H]
            nbc = nbc_ref[pl.ds(i * _CH, _CH), :]
            occ = nbc == iota_row
            ocf = jnp.where(occ, 1.0, 0.0)
            qn = jnp.dot(ocf, q, preferred_element_type=F32, precision=_PHI)
            eatt = jnp.sum(xc * qn, 1, keepdims=True)
            eatt_ref[pl.ds(i * _CH, _CH), :] = eatt
            cand = jnp.max(jnp.where(occ, eatt, -1e30), 0, keepdims=True)
            return jnp.maximum(emax_row, cand)

        emax_row = lax.fori_loop(0, _NCHUNK, pass1,
                                 jnp.full((1, BP), -1e30, F32))

        def pass2(i, carry):
            denom, racc = carry
            xc = x_ref[pl.ds(i * _CH, _CH), :H]
            nbc = nbc_ref[pl.ds(i * _CH, _CH), :]
            nbr = nbr_ref[:, pl.ds(i * _CH, _CH)]
            ocf = jnp.where(nbc == iota_row, 1.0, 0.0)
            otf = jnp.where(iota_col == nbr, 1.0, 0.0)
            eatt = eatt_ref[pl.ds(i * _CH, _CH), :]
            emax_pn = jnp.sum(ocf * emax_row, 1, keepdims=True)
            a = jnp.exp(eatt - emax_pn)
            denom = denom + jnp.dot(otf, a, preferred_element_type=F32, precision=_PHI)
            racc = racc + jnp.dot(otf, a * xc, preferred_element_type=F32, precision=_PHI)
            return denom, racc

        denom, racc = lax.fori_loop(
            0, _NCHUNK, pass2,
            (jnp.zeros((BP, 1), F32), jnp.zeros((BP, H), F32)))
        r_read = racc / jnp.maximum(denom, 1e-30)
        q_star = jnp.concatenate([q, r_read], axis=1)
    qs_ref[...] = q_star


def _set2set(x, nbc, nbr, wi, wh, bi, bh):
    return pl.pallas_call(
        _s2s_body,
        out_shape=jax.ShapeDtypeStruct((BP, 2 * H), F32),
        scratch_shapes=[pltpu.VMEM((N2, 1), F32)],
    )(x, nbc, nbr, wi, wh, bi, bh)


_TCH = 2000


def _pstats_body(pool_ref, xc_ref, wpt_ref, wpb_ref, sum_ref):
    i = pl.program_id(0)
    y = (_ddot(pool_ref[...], wpt_ref[...])
         + _ddot(xc_ref[...], wpb_ref[...]))

    @pl.when(i == 0)
    def _():
        sum_ref[...] = jnp.zeros_like(sum_ref)

    sum_ref[...] += jnp.sum(y, 0, keepdims=True)


def _pssq_body(pool_ref, xc_ref, wpt_ref, wpb_ref, sum_ref, ssq_ref):
    i = pl.program_id(0)
    y = (_ddot(pool_ref[...], wpt_ref[...])
         + _ddot(xc_ref[...], wpb_ref[...]))
    dev = y - sum_ref[...] / C

    @pl.when(i == 0)
    def _():
        ssq_ref[...] = jnp.zeros_like(ssq_ref)

    ssq_ref[...] += jnp.sum(dev * dev, 0, keepdims=True)


def _papply_body(pool_ref, xc_ref, wpt_ref, wpb_ref, sum_ref, ssq_ref,
                 gp_ref, bp_ref, ct_ref, out_ref):
    y = (_ddot(pool_ref[...], wpt_ref[...])
         + _ddot(xc_ref[...], wpb_ref[...]))
    mu = sum_ref[...] / C
    var = ssq_ref[...] / C
    pred = jnp.maximum(
        gp_ref[...] * (y - mu) * lax.rsqrt(var + EPS) + bp_ref[...], 0.0)
    eq = lax.broadcasted_iota(jnp.int32, (_TCH, NT), 1) == ct_ref[...]
    out_ref[...] = jnp.sum(jnp.where(eq, pred, 0.0), 1, keepdims=True)


def _predict(pool, xc, wpt, wpb, gp, bp, ct):
    grid = (C // _TCH,)
    tspec = pl.BlockSpec((_TCH, 2 * H), lambda i: (i, 0))
    wspec = pl.BlockSpec((2 * H, NT), lambda i: (0, 0))
    sspec = pl.BlockSpec((1, NT), lambda i: (0, 0))
    sums = pl.pallas_call(
        _pstats_body,
        grid=grid,
        in_specs=[tspec, tspec, wspec, wspec],
        out_specs=sspec,
        out_shape=jax.ShapeDtypeStruct((1, NT), F32),
    )(pool, xc, wpt, wpb)
    ssq = pl.pallas_call(
        _pssq_body,
        grid=grid,
        in_specs=[tspec, tspec, wspec, wspec, sspec],
        out_specs=sspec,
        out_shape=jax.ShapeDtypeStruct((1, NT), F32),
    )(pool, xc, wpt, wpb, sums)
    return pl.pallas_call(
        _papply_body,
        grid=grid,
        in_specs=[
            pl.BlockSpec((_TCH, 2 * H), lambda i: (i, 0)),
            pl.BlockSpec((_TCH, 2 * H), lambda i: (i, 0)),
            pl.BlockSpec((2 * H, NT), lambda i: (0, 0)),
            pl.BlockSpec((2 * H, NT), lambda i: (0, 0)),
            pl.BlockSpec((1, NT), lambda i: (0, 0)),
            pl.BlockSpec((1, NT), lambda i: (0, 0)),
            pl.BlockSpec((1, NT), lambda i: (0, 0)),
            pl.BlockSpec((1, NT), lambda i: (0, 0)),
            pl.BlockSpec((_TCH, 1), lambda i: (i, 0)),
        ],
        out_specs=pl.BlockSpec((_TCH, 1), lambda i: (i, 0)),
        out_shape=jax.ShapeDtypeStruct((C, 1), F32),
    )(pool, xc, wpt, wpb, sums, ssq, gp, bp, ct)


def _sc_gather(table, idx2, d):
    kc = idx2.shape[1]
    k = kc * 128
    mesh = plsc.VectorSubcoreMesh(core_axis_name="c", subcore_axis_name="s",
                                  num_cores=NC, num_subcores=NS)

    @functools.partial(
        pl.kernel,
        out_type=jax.ShapeDtypeStruct((NWK * k, d), F32),
        mesh=mesh,
        scratch_types=[pltpu.VMEM((kc, 128), jnp.int32),
                       pltpu.VMEM((2, 128, d), F32),
                       pltpu.SemaphoreType.DMA],
    )
    def run(table_hbm, idx_hbm, out_hbm, idx_v, rows_v, sem):
        wid = lax.axis_index("s") * NC + lax.axis_index("c")
        pltpu.sync_copy(idx_hbm.at[wid], idx_v)
        for j in range(kc):
            pltpu.async_copy(table_hbm.at[idx_v.at[j]],
                             rows_v.at[j % 2], sem).wait()
            pltpu.sync_copy(rows_v.at[j % 2],
                            out_hbm.at[pl.ds(wid * k + j * 128, 128)])

    return run(table, idx2)


def _sc_scatter_add(vals, idx2, zeros, d):
    kc = idx2.shape[1]
    k = kc * 128
    rp = NPAD // NS
    mesh = plsc.VectorSubcoreMesh(core_axis_name="c", subcore_axis_name="s",
                                  num_cores=NC, num_subcores=NS)

    @functools.partial(
        pl.kernel,
        out_type=jax.ShapeDtypeStruct((NC, NPAD, d), F32),
        mesh=mesh,
        scratch_types=[pltpu.VMEM((kc, 128), jnp.int32),
                       pltpu.VMEM((2, 128, d), F32),
                       pltpu.VMEM_SHARED((NPAD, d), F32)],
    )
    def run(vals_hbm, idx_hbm, zeros_hbm, out_hbm, idx_v, rows_v, acc_sh):
        c = lax.axis_index("c")
        s = lax.axis_index("s")
        pltpu.sync_copy(zeros_hbm.at[pl.ds(s * rp, rp)],
                        acc_sh.at[pl.ds(s * rp, rp)])
        plsc.subcore_barrier()
        wid = c * NS + s
        pltpu.sync_copy(idx_hbm.at[wid], idx_v)
        for j in range(kc):
            pltpu.sync_copy(vals_hbm.at[pl.ds(wid * k + j * 128, 128)],
                            rows_v.at[j % 2])
            pltpu.sync_copy(rows_v.at[j % 2],
                            acc_sh.at[idx_v.at[j]], add=True)
        plsc.subcore_barrier()
        pltpu.sync_copy(acc_sh.at[pl.ds(s * rp, rp)],
                        out_hbm.at[c, pl.ds(s * rp, rp)])

    return run(vals, idx2, zeros)


def _pad_idx(idx, total, fill):
    idx = idx.astype(jnp.int32)
    pad = total - idx.shape[0]
    idx = jnp.concatenate([idx, jnp.full((pad,), fill, jnp.int32)])
    return idx.reshape(NWK, total // (NWK * 128), 128)


def kernel(node, edge, edge_index, node_batch_index, coupling_index,
           coupling_type, coupling_batch_index,
           W_pre, g_pre, b_pre, We1, ge1, be1, We2, ge2, be2,
           W_root, b_root, Wih, Whh, bih, bhh, Wi, Wh, bi, bh,
           W_pred, g_pred, b_pred):
    f32 = F32
    node_p = jnp.pad(node.astype(f32), ((0, 0), (0, 16 - ND)))
    edge_p = jnp.pad(edge.astype(f32), ((0, EP - E), (0, 8 - ED)))
    wpre_p = jnp.pad(W_pre.astype(f32), ((0, 16 - ND), (0, 0)))
    we1_p = jnp.pad(We1.astype(f32), ((0, 8 - ED), (0, 0)))

    row = lambda v: v.astype(f32).reshape(1, -1)

    x0, e1 = _prep1(node_p, wpre_p, row(g_pre), row(b_pre),
                    edge_p, we1_p, row(ge1), row(be1))
    we2 = We2.astype(f32)
    mu2, sc2 = _prep2(e1, we2, row(ge2))
    be2r = row(be2)

    src2 = _pad_idx(edge_index[:, 0], EP, 0)
    dst2 = _pad_idx(edge_index[:, 1], EP, NPAD - 1)

    zeros_n = jnp.zeros((NPAD, 2 * H), f32)

    x = x0
    for _ in range(STEPS):
        xs = _sc_gather(x, src2, 2 * H)
        msg = _msg(xs, e1, we2, mu2, sc2, be2r)
        mp = _sc_scatter_add(msg, dst2, zeros_n, 2 * H)
        x = _gru(x, mp[0, :N], mp[1, :N],
                 W_root.astype(f32), row(b_root),
                 Wih.astype(f32), Whh.astype(f32), row(bih), row(bhh))

    nbi_p = jnp.concatenate([node_batch_index.astype(jnp.int32),
                             jnp.full((N2 - N,), BP - 1, jnp.int32)])
    x_p = jnp.pad(x, ((0, N2 - N), (0, 0)))
    q_star = _set2set(x_p, nbi_p.reshape(N2, 1), nbi_p.reshape(1, N2),
                      Wi.astype(f32), Wh.astype(f32), row(bi), row(bh))

    cbi2 = _pad_idx(coupling_batch_index, CP, 0)
    cix2 = _pad_idx(coupling_index.reshape(-1), C2P, 0)
    pool = _sc_gather(q_star, cbi2, 2 * H)[:C]
    xc = _sc_gather(x, cix2, 2 * H)[:2 * C, :H].reshape(C, 2 * H)

    wp = W_pred.astype(f32)
    out = _predict(pool, xc, wp[:2 * H], wp[2 * H:],
                   row(g_pred), row(b_pred),
                   coupling_type.astype(jnp.int32).reshape(C, 1))
    return out.reshape(C)


# speedup vs baseline: 1.2001x; 1.0073x over previous
import functools

import jax
import jax.numpy as jnp
from jax import lax
from jax.experimental import pallas as pl
from jax.experimental.pallas import tpu as pltpu
from jax.experimental.pallas import tpu_sc as plsc

N = 10000; E = 16000; B = 500; C = 20000
H = 64; ND = 13; ED = 5; NT = 8; STEPS = 3
EPS = 1e-5

NC, NS, NWK = 2, 16, 32
EP = 16384
NPAD = 10240
BP = 512
CP = 20480
C2P = 40960
F32 = jnp.float32
_PHI = lax.Precision.HIGHEST


def _ddot(a, b):
    return jnp.dot(a.astype(jnp.bfloat16), b.astype(jnp.bfloat16),
                   preferred_element_type=F32)


def _prep1_body(node_ref, wpre_ref, gpre_ref, bpre_ref,
                edge_ref, we1_ref, ge1_ref, be1_ref, x0_ref, e1_ref):
    y0 = _ddot(node_ref[...], wpre_ref[...])
    mu0 = jnp.sum(y0, 0, keepdims=True) / N
    d0 = y0 - mu0
    var0 = jnp.sum(d0 * d0, 0, keepdims=True) / N
    x0 = jnp.maximum(
        gpre_ref[...] * d0 * lax.rsqrt(var0 + EPS) + bpre_ref[...], 0.0)
    x0_ref[...] = jnp.concatenate([x0, jnp.zeros((N, H), F32)], axis=1)

    y1 = _ddot(edge_ref[...], we1_ref[...])
    mu1 = jnp.sum(y1, 0, keepdims=True) / E
    var1 = jnp.sum(y1 * y1, 0, keepdims=True) / E - mu1 * mu1
    e1 = jnp.maximum(
        ge1_ref[...] * (y1 - mu1) * lax.rsqrt(var1 + EPS) + be1_ref[...], 0.0)
    rid = lax.broadcasted_iota(jnp.int32, (EP, H), 0)
    e1_ref[...] = jnp.where(rid < E, e1, 0.0)


def _prep1(node_p, wpre_p, gpre, bpre, edge_p, we1_p, ge1, be1):
    return pl.pallas_call(
        _prep1_body,
        out_shape=(jax.ShapeDtypeStruct((N, 2 * H), F32),
                   jax.ShapeDtypeStruct((EP, H), F32)),
    )(node_p, wpre_p, gpre, bpre, edge_p, we1_p, ge1, be1)


_TE = 256


def _esum_body(e1_ref, we2_ref, sum_ref, esum_ref):
    i = pl.program_id(0)
    y = _ddot(e1_ref[...], we2_ref[...])

    @pl.when(i == 0)
    def _():
        sum_ref[...] = jnp.zeros_like(sum_ref)
        esum_ref[...] = jnp.zeros_like(esum_ref)

    sum_ref[...] += jnp.sum(y, 0, keepdims=True)
    esum_ref[...] += jnp.sum(e1_ref[...], 0, keepdims=True)


def _essq_body(e1_ref, we2_ref, sum_ref, ssq_ref):
    i = pl.program_id(0)
    y = _ddot(e1_ref[...], we2_ref[...])
    row = i * _TE + lax.broadcasted_iota(jnp.int32, (_TE, H * H), 0)
    dev = jnp.where(row < E, y - sum_ref[...] / E, 0.0)

    @pl.when(i == 0)
    def _():
        ssq_ref[...] = jnp.zeros_like(ssq_ref)

    ssq_ref[...] += jnp.sum(dev * dev, 0, keepdims=True)


def _prep2_body(sum_ref, ssq_ref, ge2_ref, mu_ref, sc_ref):
    mu2 = sum_ref[...] / E
    var2 = ssq_ref[...] / E
    mu_ref[...] = mu2
    sc_ref[...] = ge2_ref[...] / jnp.sqrt(var2 + EPS)


def _prep2(e1, we2, ge2):
    espec = pl.BlockSpec((_TE, H), lambda i: (i, 0))
    wspec = pl.BlockSpec((H, H * H), lambda i: (0, 0))
    sspec = pl.BlockSpec((1, H * H), lambda i: (0, 0))
    sums, _ = pl.pallas_call(
        _esum_body,
        grid=(EP // _TE,),
        in_specs=[espec, wspec],
        out_specs=(sspec, pl.BlockSpec((1, H), lambda i: (0, 0))),
        out_shape=(jax.ShapeDtypeStruct((1, H * H), F32),
                   jax.ShapeDtypeStruct((1, H), F32)),
    )(e1, we2)
    ssq = pl.pallas_call(
        _essq_body,
        grid=(EP // _TE,),
        in_specs=[espec, wspec, sspec],
        out_specs=sspec,
        out_shape=jax.ShapeDtypeStruct((1, H * H), F32),
    )(e1, we2, sums)
    return pl.pallas_call(
        _prep2_body,
        out_shape=(jax.ShapeDtypeStruct((1, H * H), F32),
                   jax.ShapeDtypeStruct((1, H * H), F32)),
    )(sums, ssq, ge2)


_TM = 256


def _msg_body(xs_ref, e1_ref, we2_ref, mu_ref, sc_ref, be2_ref, msg_ref):
    y = _ddot(e1_ref[...], we2_ref[...])
    w = (y - mu_ref[...]) * sc_ref[...] + be2_ref[...]
    xs = xs_ref[:, :H]
    m = xs[:, 0:1] * w[:, 0:H]
    for h in range(1, H):
        m = m + xs[:, h:h + 1] * w[:, h * H:(h + 1) * H]
    msg_ref[...] = jnp.concatenate(
        [m, jnp.ones((_TM, 1), F32), jnp.zeros((_TM, H - 1), F32)], axis=1)


def _msg(xs, e1, we2, mu2, sc2, be2):
    grid = (EP // _TM,)
    rspec = pl.BlockSpec((1, H * H), lambda i: (0, 0))
    return pl.pallas_call(
        _msg_body,
        grid=grid,
        in_specs=[
            pl.BlockSpec((_TM, 2 * H), lambda i: (i, 0)),
            pl.BlockSpec((_TM, H), lambda i: (i, 0)),
            pl.BlockSpec((H, H * H), lambda i: (0, 0)),
            rspec, rspec, rspec,
        ],
        out_specs=pl.BlockSpec((_TM, 2 * H), lambda i: (i, 0)),
        out_shape=jax.ShapeDtypeStruct((EP, 2 * H), F32),
    )(xs, e1, we2, mu2, sc2, be2)


def _gru_body(x_ref, p0_ref, p1_ref,
              wroot_ref, broot_ref, wih_ref, whh_ref, bih_ref, bhh_ref,
              out_ref):
    x = x_ref[:, :H]
    p = p0_ref[...] + p1_ref[...]
    deg = p[:, H:H + 1]
    aggr = p[:, :H] / jnp.maximum(deg, 1.0)
    m = jnp.maximum(
        aggr + _ddot(x, wroot_ref[...])
        + broot_ref[...], 0.0)
    gi = _ddot(m, wih_ref[...]) + bih_ref[...]
    gh = _ddot(x, whh_ref[...]) + bhh_ref[...]
    r = jax.nn.sigmoid(gi[:, :H] + gh[:, :H])
    z = jax.nn.sigmoid(gi[:, H:2 * H] + gh[:, H:2 * H])
    n = jnp.tanh(gi[:, 2 * H:] + r * gh[:, 2 * H:])
    h = (1.0 - z) * n + z * x
    out_ref[...] = jnp.concatenate([h, jnp.zeros((N, H), F32)], axis=1)


def _gru(x, p0, p1, wroot, broot, wih, whh, bih, bhh):
    return pl.pallas_call(
        _gru_body,
        out_shape=jax.ShapeDtypeStruct((N, 2 * H), F32),
    )(x, p0, p1, wroot, broot, wih, whh, bih, bhh)


N2 = 10240
_CH = 1280
_NCHUNK = N2 // _CH


def _s2s_body(x_ref, nbc_ref, nbr_ref, wi_ref, wh_ref, bi_ref, bh_ref,
              qs_ref, eatt_ref):
    q_star = jnp.zeros((BP, 2 * H), F32)
    hS = jnp.zeros((BP, H), F32)
    cS = jnp.zeros((BP, H), F32)
    iota_row = lax.broadcasted_iota(jnp.int32, (_CH, BP), 1)
    iota_col = lax.broadcasted_iota(jnp.int32, (BP, _CH), 0)
    for _ in range(STEPS):
        gates = (_ddot(q_star, wi_ref[...]) + bi_ref[...]
                 + _ddot(hS, wh_ref[...]) + bh_ref[...])
        i_g = jax.nn.sigmoid(gates[:, :H])
        f_g = jax.nn.sigmoid(gates[:, H:2 * H])
        g_g = jnp.tanh(gates[:, 2 * H:3 * H])
        o_g = jax.nn.sigmoid(gates[:, 3 * H:])
        cS = f_g * cS + i_g * g_g
        hS = o_g * jnp.tanh(cS)
        q = hS

        def pass1(i, emax_row):
            xc = x_ref[pl.ds(i * _CH, _CH), :H]
            nbc = nbc_ref[pl.ds(i * _CH, _CH), :]
            occ = nbc == iota_row
            ocf = jnp.where(occ, 1.0, 0.0)
            qn = jnp.dot(ocf, q, preferred_element_type=F32, precision=_PHI)
            eatt = jnp.sum(xc * qn, 1, keepdims=True)
            eatt_ref[pl.ds(i * _CH, _CH), :] = eatt
            cand = jnp.max(jnp.where(occ, eatt, -1e30), 0, keepdims=True)
            return jnp.maximum(emax_row, cand)

        emax_row = lax.fori_loop(0, _NCHUNK, pass1,
                                 jnp.full((1, BP), -1e30, F32))

        def pass2(i, carry):
            denom, racc = carry
            xc = x_ref[pl.ds(i * _CH, _CH), :H]
            nbc = nbc_ref[pl.ds(i * _CH, _CH), :]
            nbr = nbr_ref[:, pl.ds(i * _CH, _CH)]
            ocf = jnp.where(nbc == iota_row, 1.0, 0.0)
            otf = jnp.where(iota_col == nbr, 1.0, 0.0)
            eatt = eatt_ref[pl.ds(i * _CH, _CH), :]
            emax_pn = jnp.sum(ocf * emax_row, 1, keepdims=True)
            a = jnp.exp(eatt - emax_pn)
            denom = denom + jnp.dot(otf, a, preferred_element_type=F32, precision=_PHI)
            racc = racc + jnp.dot(otf, a * xc, preferred_element_type=F32, precision=_PHI)
            return denom, racc

        denom, racc = lax.fori_loop(
            0, _NCHUNK, pass2,
            (jnp.zeros((BP, 1), F32), jnp.zeros((BP, H), F32)))
        r_read = racc / jnp.maximum(denom, 1e-30)
        q_star = jnp.concatenate([q, r_read], axis=1)
    qs_ref[...] = q_star


def _set2set(x, nbc, nbr, wi, wh, bi, bh):
    return pl.pallas_call(
        _s2s_body,
        out_shape=jax.ShapeDtypeStruct((BP, 2 * H), F32),
        scratch_shapes=[pltpu.VMEM((N2, 1), F32)],
    )(x, nbc, nbr, wi, wh, bi, bh)


_TCH = 2000


def _pstats_body(pool_ref, xc_ref, wpt_ref, wpb_ref, sum_ref):
    i = pl.program_id(0)
    y = (_ddot(pool_ref[...], wpt_ref[...])
         + _ddot(xc_ref[...], wpb_ref[...]))

    @pl.when(i == 0)
    def _():
        sum_ref[...] = jnp.zeros_like(sum_ref)

    sum_ref[...] += jnp.sum(y, 0, keepdims=True)


def _pssq_body(pool_ref, xc_ref, wpt_ref, wpb_ref, sum_ref, ssq_ref):
    i = pl.program_id(0)
    y = (_ddot(pool_ref[...], wpt_ref[...])
         + _ddot(xc_ref[...], wpb_ref[...]))
    dev = y - sum_ref[...] / C

    @pl.when(i == 0)
    def _():
        ssq_ref[...] = jnp.zeros_like(ssq_ref)

    ssq_ref[...] += jnp.sum(dev * dev, 0, keepdims=True)


def _papply_body(pool_ref, xc_ref, wpt_ref, wpb_ref, sum_ref, ssq_ref,
                 gp_ref, bp_ref, ct_ref, out_ref):
    y = (_ddot(pool_ref[...], wpt_ref[...])
         + _ddot(xc_ref[...], wpb_ref[...]))
    mu = sum_ref[...] / C
    var = ssq_ref[...] / C
    pred = jnp.maximum(
        gp_ref[...] * (y - mu) * lax.rsqrt(var + EPS) + bp_ref[...], 0.0)
    eq = lax.broadcasted_iota(jnp.int32, (_TCH, NT), 1) == ct_ref[...]
    out_ref[...] = jnp.sum(jnp.where(eq, pred, 0.0), 1, keepdims=True)


def _predict(pool, xc, wpt, wpb, gp, bp, ct):
    grid = (C // _TCH,)
    tspec = pl.BlockSpec((_TCH, 2 * H), lambda i: (i, 0))
    wspec = pl.BlockSpec((2 * H, NT), lambda i: (0, 0))
    sspec = pl.BlockSpec((1, NT), lambda i: (0, 0))
    sums = pl.pallas_call(
        _pstats_body,
        grid=grid,
        in_specs=[tspec, tspec, wspec, wspec],
        out_specs=sspec,
        out_shape=jax.ShapeDtypeStruct((1, NT), F32),
    )(pool, xc, wpt, wpb)
    ssq = pl.pallas_call(
        _pssq_body,
        grid=grid,
        in_specs=[tspec, tspec, wspec, wspec, sspec],
        out_specs=sspec,
        out_shape=jax.ShapeDtypeStruct((1, NT), F32),
    )(pool, xc, wpt, wpb, sums)
    return pl.pallas_call(
        _papply_body,
        grid=grid,
        in_specs=[
            pl.BlockSpec((_TCH, 2 * H), lambda i: (i, 0)),
            pl.BlockSpec((_TCH, 2 * H), lambda i: (i, 0)),
            pl.BlockSpec((2 * H, NT), lambda i: (0, 0)),
            pl.BlockSpec((2 * H, NT), lambda i: (0, 0)),
            pl.BlockSpec((1, NT), lambda i: (0, 0)),
            pl.BlockSpec((1, NT), lambda i: (0, 0)),
            pl.BlockSpec((1, NT), lambda i: (0, 0)),
            pl.BlockSpec((1, NT), lambda i: (0, 0)),
            pl.BlockSpec((_TCH, 1), lambda i: (i, 0)),
        ],
        out_specs=pl.BlockSpec((_TCH, 1), lambda i: (i, 0)),
        out_shape=jax.ShapeDtypeStruct((C, 1), F32),
    )(pool, xc, wpt, wpb, sums, ssq, gp, bp, ct)


def _sc_gather(table, idx2, d):
    kc = idx2.shape[1]
    k = kc * 128
    mesh = plsc.VectorSubcoreMesh(core_axis_name="c", subcore_axis_name="s",
                                  num_cores=NC, num_subcores=NS)

    @functools.partial(
        pl.kernel,
        out_type=jax.ShapeDtypeStruct((NWK * k, d), F32),
        mesh=mesh,
        scratch_types=[pltpu.VMEM((kc, 128), jnp.int32),
                       pltpu.VMEM((2, 128, d), F32),
                       pltpu.SemaphoreType.DMA],
    )
    def run(table_hbm, idx_hbm, out_hbm, idx_v, rows_v, sem):
        wid = lax.axis_index("s") * NC + lax.axis_index("c")
        pltpu.sync_copy(idx_hbm.at[wid], idx_v)
        descs = [None, None]
        descs[0] = pltpu.async_copy(table_hbm.at[idx_v.at[0]],
                                    rows_v.at[0], sem)
        for j in range(kc):
            if j + 1 < kc:
                descs[(j + 1) % 2] = pltpu.async_copy(
                    table_hbm.at[idx_v.at[j + 1]], rows_v.at[(j + 1) % 2], sem)
            descs[j % 2].wait()
            pltpu.sync_copy(rows_v.at[j % 2],
                            out_hbm.at[pl.ds(wid * k + j * 128, 128)])

    return run(table, idx2)


def _sc_scatter_add(vals, idx2, zeros, d):
    kc = idx2.shape[1]
    k = kc * 128
    rp = NPAD // NS
    mesh = plsc.VectorSubcoreMesh(core_axis_name="c", subcore_axis_name="s",
                                  num_cores=NC, num_subcores=NS)

    @functools.partial(
        pl.kernel,
        out_type=jax.ShapeDtypeStruct((NC, NPAD, d), F32),
        mesh=mesh,
        scratch_types=[pltpu.VMEM((kc, 128), jnp.int32),
                       pltpu.VMEM((2, 128, d), F32),
                       pltpu.VMEM_SHARED((NPAD, d), F32),
                       pltpu.SemaphoreType.DMA],
    )
    def run(vals_hbm, idx_hbm, zeros_hbm, out_hbm, idx_v, rows_v, acc_sh, sem):
        c = lax.axis_index("c")
        s = lax.axis_index("s")
        pltpu.sync_copy(zeros_hbm.at[pl.ds(s * rp, rp)],
                        acc_sh.at[pl.ds(s * rp, rp)])
        plsc.subcore_barrier()
        wid = c * NS + s
        pltpu.sync_copy(idx_hbm.at[wid], idx_v)
        descs = [None, None]
        descs[0] = pltpu.async_copy(vals_hbm.at[pl.ds(wid * k, 128)],
                                    rows_v.at[0], sem)
        for j in range(kc):
            if j + 1 < kc:
                descs[(j + 1) % 2] = pltpu.async_copy(
                    vals_hbm.at[pl.ds(wid * k + (j + 1) * 128, 128)],
                    rows_v.at[(j + 1) % 2], sem)
            descs[j % 2].wait()
            pltpu.sync_copy(rows_v.at[j % 2],
                            acc_sh.at[idx_v.at[j]], add=True)
        plsc.subcore_barrier()
        pltpu.sync_copy(acc_sh.at[pl.ds(s * rp, rp)],
                        out_hbm.at[c, pl.ds(s * rp, rp)])

    return run(vals, idx2, zeros)


def _pad_idx(idx, total, fill):
    idx = idx.astype(jnp.int32)
    pad = total - idx.shape[0]
    idx = jnp.concatenate([idx, jnp.full((pad,), fill, jnp.int32)])
    return idx.reshape(NWK, total // (NWK * 128), 128)


def kernel(node, edge, edge_index, node_batch_index, coupling_index,
           coupling_type, coupling_batch_index,
           W_pre, g_pre, b_pre, We1, ge1, be1, We2, ge2, be2,
           W_root, b_root, Wih, Whh, bih, bhh, Wi, Wh, bi, bh,
           W_pred, g_pred, b_pred):
    f32 = F32
    node_p = jnp.pad(node.astype(f32), ((0, 0), (0, 16 - ND)))
    edge_p = jnp.pad(edge.astype(f32), ((0, EP - E), (0, 8 - ED)))
    wpre_p = jnp.pad(W_pre.astype(f32), ((0, 16 - ND), (0, 0)))
    we1_p = jnp.pad(We1.astype(f32), ((0, 8 - ED), (0, 0)))

    row = lambda v: v.astype(f32).reshape(1, -1)

    x0, e1 = _prep1(node_p, wpre_p, row(g_pre), row(b_pre),
                    edge_p, we1_p, row(ge1), row(be1))
    we2 = We2.astype(f32)
    mu2, sc2 = _prep2(e1, we2, row(ge2))
    be2r = row(be2)

    src2 = _pad_idx(edge_index[:, 0], EP, 0)
    dst2 = _pad_idx(edge_index[:, 1], EP, NPAD - 1)

    zeros_n = jnp.zeros((NPAD, 2 * H), f32)

    x = x0
    for _ in range(STEPS):
        xs = _sc_gather(x, src2, 2 * H)
        msg = _msg(xs, e1, we2, mu2, sc2, be2r)
        mp = _sc_scatter_add(msg, dst2, zeros_n, 2 * H)
        x = _gru(x, mp[0, :N], mp[1, :N],
                 W_root.astype(f32), row(b_root),
                 Wih.astype(f32), Whh.astype(f32), row(bih), row(bhh))

    nbi_p = jnp.concatenate([node_batch_index.astype(jnp.int32),
                             jnp.full((N2 - N,), BP - 1, jnp.int32)])
    x_p = jnp.pad(x, ((0, N2 - N), (0, 0)))
    q_star = _set2set(x_p, nbi_p.reshape(N2, 1), nbi_p.reshape(1, N2),
                      Wi.astype(f32), Wh.astype(f32), row(bi), row(bh))

    cbi2 = _pad_idx(coupling_batch_index, CP, 0)
    cix2 = _pad_idx(coupling_index.reshape(-1), C2P, 0)
    pool = _sc_gather(q_star, cbi2, 2 * H)[:C]
    xc = _sc_gather(x, cix2, 2 * H)[:2 * C, :H].reshape(C, 2 * H)

    wp = W_pred.astype(f32)
    out = _predict(pool, xc, wp[:2 * H], wp[2 * H:],
                   row(g_pred), row(b_pred),
                   coupling_type.astype(jnp.int32).reshape(C, 1))
    return out.reshape(C)
